# baseline (speedup 1.0000x reference)
import jax
import jax.numpy as jnp
from jax import lax
from jax.experimental import pallas as pl
from jax.experimental.pallas import tpu as pltpu
from jax.experimental.pallas import tpu_sc as plsc

N = 10000
E = 320000
D = 128
NV = D // 16

NC = 2
NS = 16
NW = NC * NS
RPW = 312
RLAST = N - (NW - 1) * RPW

SCN = 2000
NCHK = E // SCN
SEG = 8
NSEG = NCHK // SEG
WC = 80
LCAP = 8256
RB = 3


def _mm_body(x_ref, w_ref, o_ref):
    o_ref[...] = jnp.dot(x_ref[...], w_ref[...],
                         preferred_element_type=jnp.float32)


def _matmul(x, W, bm=2000):
    n, k = x.shape
    m = W.shape[1]
    return pl.pallas_call(
        _mm_body,
        grid=(n // bm,),
        in_specs=[pl.BlockSpec((bm, k), lambda i: (i, 0)),
                  pl.BlockSpec((k, m), lambda i: (0, 0))],
        out_specs=pl.BlockSpec((bm, m), lambda i: (i, 0)),
        out_shape=jax.ShapeDtypeStruct((n, m), jnp.float32),
    )(x, W)


def _fused_body(p_ref, b_ref, w_ref, o_ref):
    h = jnp.maximum(p_ref[...] + b_ref[...], 0.0)
    o_ref[...] = jnp.dot(h, w_ref[...], preferred_element_type=jnp.float32)


def _fused_relu_mm(p, b, W, bm=2000):
    n, k = p.shape
    m = W.shape[1]
    return pl.pallas_call(
        _fused_body,
        grid=(n // bm,),
        in_specs=[pl.BlockSpec((bm, k), lambda i: (i, 0)),
                  pl.BlockSpec((1, k), lambda i: (0, 0)),
                  pl.BlockSpec((k, m), lambda i: (0, 0))],
        out_specs=pl.BlockSpec((bm, m), lambda i: (i, 0)),
        out_shape=jax.ShapeDtypeStruct((n, m), jnp.float32),
    )(p, b.reshape(1, k), W)


def _bias_body(p_ref, b_ref, o_ref):
    o_ref[...] = p_ref[...] + b_ref[...]


def _add_bias(p, b, bm=2000):
    n, k = p.shape
    return pl.pallas_call(
        _bias_body,
        grid=(n // bm,),
        in_specs=[pl.BlockSpec((bm, k), lambda i: (i, 0)),
                  pl.BlockSpec((1, k), lambda i: (0, 0))],
        out_specs=pl.BlockSpec((bm, k), lambda i: (i, 0)),
        out_shape=jax.ShapeDtypeStruct((n, k), jnp.float32),
    )(p, b.reshape(1, k))


def _agg_body(h_hbm, src_hbm, dst_hbm, w_hbm, out_hbm,
              sbuf, dbuf, wbuf, lsrc, ldl, lw, rowbuf, acc, isem, gsem):
    c = lax.axis_index("c")
    s = lax.axis_index("s")
    wid = c * NS + s

    lo = wid * RPW
    rows_own = jnp.where(wid == NW - 1, RLAST, RPW).astype(jnp.int32)
    hi = lo + rows_own

    def _zrow(r, carry):
        zero = jnp.zeros((16,), jnp.float32)
        for j in range(NV):
            acc[r, pl.ds(j * 16, 16)] = zero
        return carry
    lax.fori_loop(0, RLAST, _zrow, None)

    def fire_scan(k):
        sl = lax.rem(k, 2)
        base = pl.multiple_of(k * SCN, 8)
        sb = pl.multiple_of(sl * SCN, 8)
        pltpu.async_copy(src_hbm.at[pl.ds(base, SCN)],
                         sbuf.at[pl.ds(sb, SCN)], isem.at[sl])
        pltpu.async_copy(dst_hbm.at[pl.ds(base, SCN)],
                         dbuf.at[pl.ds(sb, SCN)], isem.at[sl])
        pltpu.async_copy(w_hbm.at[pl.ds(base, SCN)],
                         wbuf.at[pl.ds(sb, SCN)], isem.at[sl])

    def wait_scan(k):
        sl = lax.rem(k, 2)
        sb = pl.multiple_of(sl * SCN, 8)
        pltpu.make_async_copy(src_hbm.at[pl.ds(0, SCN)],
                              sbuf.at[pl.ds(sb, SCN)], isem.at[sl]).wait()
        pltpu.make_async_copy(dst_hbm.at[pl.ds(0, SCN)],
                              dbuf.at[pl.ds(sb, SCN)], isem.at[sl]).wait()
        pltpu.make_async_copy(w_hbm.at[pl.ds(0, SCN)],
                              wbuf.at[pl.ds(sb, SCN)], isem.at[sl]).wait()

    def fire_gather(p):
        ri = lax.rem(p, RB)
        pltpu.async_copy(h_hbm.at[lsrc.at[pl.ds(p * WC, WC)]], rowbuf.at[ri],
                         gsem.at[ri])

    def wait_gather(p):
        ri = lax.rem(p, RB)
        pltpu.make_async_copy(h_hbm.at[lsrc.at[pl.ds(0, WC)]], rowbuf.at[ri],
                              gsem.at[ri]).wait()

    def process_window(p):
        ri = lax.rem(p, RB)

        def _g16(q, carry):
            base = p * WC + q * 16
            w16 = lw[pl.ds(base, 16)]
            l16 = ldl[pl.ds(base, 16)]
            for i in range(16):
                wv = jnp.full((16,), w16[i], jnp.float32)
                lrow = l16[i]
                r = q * 16 + i
                for j in range(NV):
                    plsc.addupdate(
                        acc.at[lrow, pl.ds(j * 16, 16)],
                        rowbuf[ri, r, pl.ds(j * 16, 16)] * wv)
            return carry
        lax.fori_loop(0, WC // 16, _g16, None)

    fire_scan(0)
    fire_scan(1)

    def _segment(sg, off):
        def _chunk(kk, off):
            k = sg * SEG + kk
            wait_scan(k)
            sl = lax.rem(k, 2)

            def _step(q, off):
                d16 = dbuf[pl.ds(sl * SCN + q * 16, 16)]
                s16 = sbuf[pl.ds(sl * SCN + q * 16, 16)]
                w16 = wbuf[pl.ds(sl * SCN + q * 16, 16)]
                keep = jnp.logical_and(d16 >= lo, d16 < hi)
                plsc.store_compressed(lsrc.at[pl.ds(off, 16)], s16, mask=keep)
                plsc.store_compressed(lw.at[pl.ds(off, 16)], w16, mask=keep)
                plsc.store_compressed(ldl.at[pl.ds(off, 16)], d16 - lo,
                                      mask=keep)
                cnt = plsc.all_reduce_population_count(keep)
                cnt = cnt[0] if getattr(cnt, "ndim", 0) else cnt
                return lax.min(off + cnt, jnp.int32(LCAP - 16))
            off = lax.fori_loop(0, SCN // 16, _step, off)

            @pl.when(k + 2 < NCHK)
            def _():
                fire_scan(k + 2)
            return off
        off = lax.fori_loop(0, SEG, _chunk, off)

        n_full = lax.div(off, jnp.int32(WC))

        @pl.when(n_full >= 1)
        def _():
            fire_gather(0)

        @pl.when(n_full >= 2)
        def _():
            fire_gather(1)

        @pl.when(n_full >= 3)
        def _():
            fire_gather(2)

        @pl.loop(0, n_full)
        def _win(p):
            wait_gather(p)
            process_window(p)

            @pl.when(p + 3 < n_full)
            def _():
                fire_gather(p + 3)

        rem = off - n_full * WC

        @pl.when(n_full > 0)
        def _carry_rem():
            tailb = pl.multiple_of(n_full * WC, 8)
            for t in range(WC // 16):
                v_s = lsrc[pl.ds(tailb + t * 16, 16)]
                v_w = lw[pl.ds(tailb + t * 16, 16)]
                v_l = ldl[pl.ds(tailb + t * 16, 16)]
                lsrc[pl.ds(t * 16, 16)] = v_s
                lw[pl.ds(t * 16, 16)] = v_w
                ldl[pl.ds(t * 16, 16)] = v_l
        return rem
    off = lax.fori_loop(0, NSEG, _segment, jnp.int32(0))

    @pl.when(off > 0)
    def _drain():
        zi = jnp.zeros((16,), jnp.int32)
        zf = jnp.zeros((16,), jnp.float32)
        for t in range(6):
            lsrc[pl.ds(off + t * 16, 16)] = zi
            ldl[pl.ds(off + t * 16, 16)] = zi
            lw[pl.ds(off + t * 16, 16)] = zf
        fire_gather(0)
        wait_gather(0)
        process_window(0)

    for i in range(RPW // 104):
        pltpu.sync_copy(acc.at[pl.ds(i * 104, 104)],
                        out_hbm.at[pl.ds(lo + i * 104, 104)])

    @pl.when(wid == NW - 1)
    def _tail():
        pltpu.sync_copy(acc.at[pl.ds(RPW, RLAST - RPW)],
                        out_hbm.at[pl.ds(lo + RPW, RLAST - RPW)])


def _sc_aggregate(h, src, dst, w):
    mesh = plsc.VectorSubcoreMesh(core_axis_name="c", subcore_axis_name="s")
    run = pl.kernel(
        _agg_body,
        out_type=jax.ShapeDtypeStruct((N, D), jnp.float32),
        mesh=mesh,
        compiler_params=pltpu.CompilerParams(needs_layout_passes=False),
        scratch_types=[
            pltpu.VMEM((2 * SCN,), jnp.int32),
            pltpu.VMEM((2 * SCN,), jnp.int32),
            pltpu.VMEM((2 * SCN,), jnp.float32),
            pltpu.VMEM((LCAP,), jnp.int32),
            pltpu.VMEM((LCAP,), jnp.int32),
            pltpu.VMEM((LCAP,), jnp.float32),
            pltpu.VMEM((RB, WC, D), jnp.float32),
            pltpu.VMEM((RLAST, D), jnp.float32),
            pltpu.SemaphoreType.DMA((2,)),
            pltpu.SemaphoreType.DMA((RB,)),
        ],
    )
    return run(h, src, dst, w)


@jax.jit
def kernel(x, edge_index, edge_weight, W1, b1, W2, b2):
    src = edge_index[0]
    dst = edge_index[1]
    h1 = _matmul(x, W1)
    a1 = _sc_aggregate(h1, src, dst, edge_weight)
    h2 = _fused_relu_mm(a1, b1, W2)
    a2 = _sc_aggregate(h2, src, dst, edge_weight)
    return _add_bias(a2, b2)
